# Initial kernel scaffold; baseline (speedup 1.0000x reference)
import jax
import jax.numpy as jnp
from jax.experimental import pallas as pl


def kernel(x, gate_w, shared_gate, shared_up, shared_down, routed_gate, routed_up, routed_down):
    raise NotImplementedError("write your pallas kernel here")


# speedup vs baseline: 2.1119x; 2.1119x over previous
import functools

import jax
import jax.numpy as jnp
from jax.experimental import pallas as pl
from jax.experimental.pallas import tpu as pltpu

B, T, D = 1, 2048, 1024
INTER = 512
E = 8
ROUTE_SCALE = 2.5

BT = 1024


def _ffn(xbh, g_ref, u_ref, d_ref):
    dn = (((1,), (1,)), ((), ()))
    g = jax.lax.dot_general(xbh, g_ref[0], dn, preferred_element_type=jnp.float32)
    u = jax.lax.dot_general(xbh, u_ref[0], dn, preferred_element_type=jnp.float32)
    h = (g * (1.0 / (1.0 + jnp.exp(-g)))) * u
    return jax.lax.dot_general(h.astype(jnp.bfloat16), d_ref[0], dn,
                               preferred_element_type=jnp.float32)


def _moe_body(x_ref, gate_ref, sg_ref, su_ref, sd_ref, rg_ref, ru_ref, rd_ref,
              out_ref, aux_ref, i1_ref, i2_ref, w1_ref, w2_ref, sums_ref):
    t = pl.program_id(0)
    e = pl.program_id(1)

    @pl.when(e == 0)
    def _router_and_shared():
        xb = x_ref[:]
        logits = jax.lax.dot_general(
            xb, gate_ref[:], (((1,), (1,)), ((), ())),
            preferred_element_type=jnp.float32) * ROUTE_SCALE
        mx = jnp.max(logits, axis=1, keepdims=True)
        ex = jnp.exp(logits - mx)
        scores = ex / jnp.sum(ex, axis=1, keepdims=True)
        iota8 = jax.lax.broadcasted_iota(jnp.int32, (BT, E), 1)
        m1 = jnp.max(scores, axis=1, keepdims=True)
        i1 = jnp.min(jnp.where(scores == m1, iota8, E), axis=1, keepdims=True)
        masked = jnp.where(iota8 == i1, -jnp.inf, scores)
        m2 = jnp.max(masked, axis=1, keepdims=True)
        i2 = jnp.min(jnp.where(masked == m2, iota8, E), axis=1, keepdims=True)
        s = m1 + m2
        i1_ref[:] = i1
        i2_ref[:] = i2
        w1_ref[:] = m1 / s
        w2_ref[:] = m2 / s
        onehot = ((iota8 == i1) | (iota8 == i2)).astype(jnp.float32)

        @pl.when(t == 0)
        def _():
            sums_ref[:] = jnp.zeros_like(sums_ref)

        sums_ref[0:1, :] += jnp.sum(onehot, axis=0, keepdims=True)
        sums_ref[1:2, :] += jnp.sum(scores, axis=0, keepdims=True)

        out_ref[:] = _ffn(xb.astype(jnp.bfloat16), sg_ref, su_ref, sd_ref)

    @pl.when(e > 0)
    def _routed():
        eidx = e - 1
        y = _ffn(x_ref[:].astype(jnp.bfloat16), rg_ref, ru_ref, rd_ref)
        we = (jnp.where(i1_ref[:] == eidx, w1_ref[:], 0.0)
              + jnp.where(i2_ref[:] == eidx, w2_ref[:], 0.0))
        out_ref[:] += we * y

    aux_ref[:] = (E / (T * T)) * jnp.sum(
        sums_ref[0:1, :] * sums_ref[1:2, :], axis=1, keepdims=True)


@jax.jit
def kernel(x, gate_w, shared_gate, shared_up, shared_down,
           routed_gate, routed_up, routed_down):
    flat = x.reshape(T, D)
    bf = jnp.bfloat16
    grid = (T // BT, E + 1)

    def wspec(shape3):
        return pl.BlockSpec(
            (1,) + shape3, lambda t, e: (jnp.maximum(e - 1, 0), 0, 0))

    out, aux = pl.pallas_call(
        _moe_body,
        grid=grid,
        in_specs=[
            pl.BlockSpec((BT, D), lambda t, e: (t, 0)),
            pl.BlockSpec((E, D), lambda t, e: (0, 0)),
            pl.BlockSpec((1, INTER, D), lambda t, e: (0, 0, 0)),
            pl.BlockSpec((1, INTER, D), lambda t, e: (0, 0, 0)),
            pl.BlockSpec((1, D, INTER), lambda t, e: (0, 0, 0)),
            wspec((INTER, D)),
            wspec((INTER, D)),
            wspec((D, INTER)),
        ],
        out_specs=[
            pl.BlockSpec((BT, D), lambda t, e: (t, 0)),
            pl.BlockSpec((1, 1), lambda t, e: (0, 0)),
        ],
        out_shape=[
            jax.ShapeDtypeStruct((T, D), jnp.float32),
            jax.ShapeDtypeStruct((1, 1), jnp.float32),
        ],
        scratch_shapes=[
            pltpu.VMEM((BT, 1), jnp.int32),
            pltpu.VMEM((BT, 1), jnp.int32),
            pltpu.VMEM((BT, 1), jnp.float32),
            pltpu.VMEM((BT, 1), jnp.float32),
            pltpu.VMEM((2, E), jnp.float32),
        ],
        compiler_params=pltpu.CompilerParams(
            dimension_semantics=("arbitrary", "arbitrary")),
    )(flat, gate_w, shared_gate.astype(bf), shared_up.astype(bf),
      shared_down.astype(bf), routed_gate.astype(bf), routed_up.astype(bf),
      routed_down.astype(bf))
    return out.reshape(B, T, D), aux[0, 0]


# speedup vs baseline: 2.1326x; 1.0098x over previous
import functools

import jax
import jax.numpy as jnp
from jax.experimental import pallas as pl
from jax.experimental.pallas import tpu as pltpu

B, T, D = 1, 2048, 1024
INTER = 512
E = 8
ROUTE_SCALE = 2.5

BT = 2048


def _ffn(xbh, g_ref, u_ref, d_ref):
    dn = (((1,), (1,)), ((), ()))
    g = jax.lax.dot_general(xbh, g_ref[0], dn, preferred_element_type=jnp.float32)
    u = jax.lax.dot_general(xbh, u_ref[0], dn, preferred_element_type=jnp.float32)
    h = (g * (1.0 / (1.0 + jnp.exp(-g)))) * u
    return jax.lax.dot_general(h.astype(jnp.bfloat16), d_ref[0], dn,
                               preferred_element_type=jnp.float32)


def _moe_body(x_ref, gate_ref, sg_ref, su_ref, sd_ref, rg_ref, ru_ref, rd_ref,
              out_ref, aux_ref, i1_ref, i2_ref, w1_ref, w2_ref, sums_ref):
    t = pl.program_id(0)
    e = pl.program_id(1)

    @pl.when(e == 0)
    def _router_and_shared():
        xb = x_ref[:]
        logits = jax.lax.dot_general(
            xb, gate_ref[:], (((1,), (1,)), ((), ())),
            preferred_element_type=jnp.float32) * ROUTE_SCALE
        mx = jnp.max(logits, axis=1, keepdims=True)
        ex = jnp.exp(logits - mx)
        scores = ex / jnp.sum(ex, axis=1, keepdims=True)
        iota8 = jax.lax.broadcasted_iota(jnp.int32, (BT, E), 1)
        m1 = jnp.max(scores, axis=1, keepdims=True)
        i1 = jnp.min(jnp.where(scores == m1, iota8, E), axis=1, keepdims=True)
        masked = jnp.where(iota8 == i1, -jnp.inf, scores)
        m2 = jnp.max(masked, axis=1, keepdims=True)
        i2 = jnp.min(jnp.where(masked == m2, iota8, E), axis=1, keepdims=True)
        s = m1 + m2
        i1_ref[:] = i1
        i2_ref[:] = i2
        w1_ref[:] = m1 / s
        w2_ref[:] = m2 / s
        onehot = ((iota8 == i1) | (iota8 == i2)).astype(jnp.float32)

        @pl.when(t == 0)
        def _():
            sums_ref[:] = jnp.zeros_like(sums_ref)

        sums_ref[0:1, :] += jnp.sum(onehot, axis=0, keepdims=True)
        sums_ref[1:2, :] += jnp.sum(scores, axis=0, keepdims=True)

        out_ref[:] = _ffn(xb.astype(jnp.bfloat16), sg_ref, su_ref, sd_ref)

    @pl.when(e > 0)
    def _routed():
        eidx = e - 1
        y = _ffn(x_ref[:].astype(jnp.bfloat16), rg_ref, ru_ref, rd_ref)
        we = (jnp.where(i1_ref[:] == eidx, w1_ref[:], 0.0)
              + jnp.where(i2_ref[:] == eidx, w2_ref[:], 0.0))
        out_ref[:] += we * y

    aux_ref[:] = (E / (T * T)) * jnp.sum(
        sums_ref[0:1, :] * sums_ref[1:2, :], axis=1, keepdims=True)


@jax.jit
def kernel(x, gate_w, shared_gate, shared_up, shared_down,
           routed_gate, routed_up, routed_down):
    flat = x.reshape(T, D)
    bf = jnp.bfloat16
    grid = (T // BT, E + 1)

    def wspec(shape3):
        return pl.BlockSpec(
            (1,) + shape3, lambda t, e: (jnp.maximum(e - 1, 0), 0, 0))

    out, aux = pl.pallas_call(
        _moe_body,
        grid=grid,
        in_specs=[
            pl.BlockSpec((BT, D), lambda t, e: (t, 0)),
            pl.BlockSpec((E, D), lambda t, e: (0, 0)),
            pl.BlockSpec((1, INTER, D), lambda t, e: (0, 0, 0)),
            pl.BlockSpec((1, INTER, D), lambda t, e: (0, 0, 0)),
            pl.BlockSpec((1, D, INTER), lambda t, e: (0, 0, 0)),
            wspec((INTER, D)),
            wspec((INTER, D)),
            wspec((D, INTER)),
        ],
        out_specs=[
            pl.BlockSpec((BT, D), lambda t, e: (t, 0)),
            pl.BlockSpec((1, 1), lambda t, e: (0, 0)),
        ],
        out_shape=[
            jax.ShapeDtypeStruct((T, D), jnp.float32),
            jax.ShapeDtypeStruct((1, 1), jnp.float32),
        ],
        scratch_shapes=[
            pltpu.VMEM((BT, 1), jnp.int32),
            pltpu.VMEM((BT, 1), jnp.int32),
            pltpu.VMEM((BT, 1), jnp.float32),
            pltpu.VMEM((BT, 1), jnp.float32),
            pltpu.VMEM((2, E), jnp.float32),
        ],
        compiler_params=pltpu.CompilerParams(
            dimension_semantics=("arbitrary", "arbitrary")),
    )(flat, gate_w, shared_gate.astype(bf), shared_up.astype(bf),
      shared_down.astype(bf), routed_gate.astype(bf), routed_up.astype(bf),
      routed_down.astype(bf))
    return out.reshape(B, T, D), aux[0, 0]
